# baseline (speedup 1.0000x reference)
import functools

import jax
import jax.numpy as jnp
from jax import lax
from jax.experimental import pallas as pl
from jax.experimental.pallas import tpu as pltpu
from jax.experimental.pallas import tpu_sc as plsc

N = 10000
NTILE = 16
RPT = N // NTILE


def _embed_body(x_ref, w_ref, b_ref, o_ref):
    o_ref[...] = jnp.maximum(
        jnp.dot(x_ref[...], w_ref[...], preferred_element_type=jnp.float32)
        + b_ref[...],
        0.0,
    )


def _embed(x, W, b):
    n, d = x.shape
    h = W.shape[1]
    blk = 2000
    return pl.pallas_call(
        _embed_body,
        grid=(n // blk,),
        in_specs=[
            pl.BlockSpec((blk, d), lambda i: (i, 0)),
            pl.BlockSpec((d, h), lambda i: (0, 0)),
            pl.BlockSpec((1, h), lambda i: (0, 0)),
        ],
        out_specs=pl.BlockSpec((blk, h), lambda i: (i, 0)),
        out_shape=jax.ShapeDtypeStruct((n, h), jnp.float32),
    )(x, W, b.reshape(1, h))


def _spmm_half_kernel(src_ref, r_ref, c_ref, u_ref, acc, gbuf, cbuf, rbuf,
                      is0, is1, is2, is3, gs0, gs1, ss0, ss1):
    c = lax.axis_index("c")
    s = lax.axis_index("s")
    w = c * NTILE + s
    ntot = (r_ref.shape[0] // 128) // 32
    isems = (is0, is1, is2, is3)
    gsems = (gs0, gs1)
    ssems = (ss0, ss1)

    z16 = jnp.zeros((1, 16), jnp.float32)
    zbuf = gbuf.at[0]

    @pl.loop(0, 128)
    def _(i):
        for k8 in range(8):
            zbuf[pl.ds(i, 1), pl.ds(k8 * 16, 16)] = z16

    for j in range(5):
        pltpu.sync_copy(zbuf, acc.at[pl.ds(s * 640 + j * 128, 128)])
    plsc.subcore_barrier()

    def do_idx(k, p):
        off = (w * ntot + k) * 128
        pltpu.sync_copy(c_ref.at[pl.ds(off, 128)], cbuf.at[p])
        pltpu.sync_copy(r_ref.at[pl.ds(off, 128)], rbuf.at[p])

    def do_gather(p):
        pltpu.sync_copy(src_ref.at[cbuf.at[p]], gbuf.at[p])

    def sc_start(p):
        pltpu.async_copy(gbuf.at[p], acc.at[rbuf.at[p]], ssems[p], add=True)

    def sc_wait(p):
        pltpu.make_async_copy(gbuf.at[p], acc.at[rbuf.at[p]],
                              ssems[p]).wait()

    @pl.loop(0, ntot)
    def _(j):
        do_idx(j, 0)
        do_gather(0)
        sc_start(0)
        sc_wait(0)

    plsc.subcore_barrier()
    pltpu.sync_copy(acc.at[pl.ds(s * 640, 640)], u_ref.at[w])


def _spmm(rows, cols, src):
    mesh = plsc.VectorSubcoreMesh(core_axis_name="c", subcore_axis_name="s")
    k = pl.kernel(
        _spmm_half_kernel,
        out_type=jax.ShapeDtypeStruct((2 * NTILE, 640, 128), jnp.float32),
        mesh=mesh,
        scratch_types=[
            pltpu.VMEM_SHARED((NPAD, 128), jnp.float32),
            pltpu.VMEM((2, 128, 128), jnp.float32),
            pltpu.VMEM((4, 128), jnp.int32),
            pltpu.VMEM((4, 128), jnp.int32),
        ] + [pltpu.SemaphoreType.DMA] * 8,
    )
    u = k(src, rows, cols).reshape(2, NPAD, 128)
    return u[0] + u[1]


def _deg_kernel(r1_ref, c1_ref, r2_ref, c2_ref, out_ref,
                a0, a1, a2, a3, ones, ibuf, zbuf):
    c = lax.axis_index("c")
    s = lax.axis_index("s")
    accs = (a0, a1, a2, a3)

    z16 = jnp.zeros((16,), jnp.float32)
    o16 = jnp.full((16,), 1.0, jnp.float32)
    for k8 in range(8):
        ones[pl.ds(k8 * 16, 16)] = o16
    for k8 in range(40):
        zbuf[pl.ds(k8 * 16, 16)] = z16

    lo = s * 640
    for acc in accs:
        pltpu.sync_copy(zbuf, acc.at[pl.ds(lo, 640)])

    plsc.subcore_barrier()

    for m, idx_hbm in enumerate((r1_ref, c1_ref, r2_ref, c2_ref)):
        nch = idx_hbm.shape[0] // 128
        w = c * NTILE + s
        jlo = w * nch // 32
        jhi = (w + 1) * nch // 32

        @pl.loop(jlo, jhi)
        def _(j):
            pltpu.sync_copy(idx_hbm.at[pl.ds(j * 128, 128)], ibuf)
            pltpu.sync_copy(ones, accs[m].at[ibuf], add=True)

    plsc.subcore_barrier()

    for m in range(4):
        pltpu.sync_copy(accs[m].at[pl.ds(lo, 640)],
                        out_ref.at[c * 4 + m, 0, pl.ds(lo, 640)])


NPAD = 10240


def _degrees(r1, c1, r2, c2):
    mesh = plsc.VectorSubcoreMesh(core_axis_name="c", subcore_axis_name="s")
    k = pl.kernel(
        _deg_kernel,
        out_type=jax.ShapeDtypeStruct((8, 1, NPAD), jnp.float32),
        mesh=mesh,
        scratch_types=[
            pltpu.VMEM_SHARED((NPAD,), jnp.float32),
            pltpu.VMEM_SHARED((NPAD,), jnp.float32),
            pltpu.VMEM_SHARED((NPAD,), jnp.float32),
            pltpu.VMEM_SHARED((NPAD,), jnp.float32),
            pltpu.VMEM((128,), jnp.float32),
            pltpu.VMEM((128,), jnp.int32),
            pltpu.VMEM((640,), jnp.float32),
        ],
    )
    d = k(r1, c1, r2, c2).reshape(2, 4, NPAD)[:, :, :N]
    return d[0] + d[1]


def kernel(x, edge_index, edge_index2, W_embed, b_embed, gamma0, beta0,
           W_final, b_final):
    n = x.shape[0]
    r1, c1 = edge_index[0], edge_index[1]
    r2, c2 = edge_index2[0], edge_index2[1]
    deg = _degrees(r1, c1, r2, c2)

    def pad_e(e):
        epad = -e.shape[0] % 16384
        return jnp.pad(e, (0, epad), constant_values=NPAD - 1)

    r1p, c1p, r2p, c2p = pad_e(r1), pad_e(c1), pad_e(r2), pad_e(c2)
    inv = jnp.where(deg > 0, jax.lax.rsqrt(deg), 0.0)
    dr1, dc1, dr2, dc2 = inv[0], inv[1], inv[2], inv[3]
    h = _embed(x, W_embed, b_embed)

    def pass_(y, lo):
        ya = jnp.pad(y[:, lo:lo + 128] * dc1[:, None], ((0, NPAD - n), (0, 0)))
        yb = jnp.pad(y[:, lo:lo + 128] * dc2[:, None], ((0, NPAD - n), (0, 0)))
        u1 = _spmm(r1p, c1p, ya)[:n]
        u2 = _spmm(r2p, c2p, yb)[:n]
        return u1 * dr1[:, None], u2 * dr2[:, None]

    s1h, s2h = pass_(h, 0)
    h1 = jnp.concatenate([s1h, s2h], axis=1)
    m = jnp.mean(h1, axis=0)
    v = jnp.var(h1, axis=0)
    h1 = (h1 - m) / jnp.sqrt(v + 1e-5) * gamma0 + beta0
    v1a, v2a = pass_(h1, 0)
    v1b, v2b = pass_(h1, 128)
    h2 = jnp.concatenate([v1a, v1b, v2a, v2b], axis=1)
    hj = jnp.concatenate([h, h1, h2], axis=1)
    out = hj @ W_final + b_final
    return jax.nn.log_softmax(out, axis=1)


# speedup vs baseline: 2.1968x; 2.1968x over previous
import functools

import jax
import jax.numpy as jnp
from jax import lax
from jax.experimental import pallas as pl
from jax.experimental.pallas import tpu as pltpu
from jax.experimental.pallas import tpu_sc as plsc

N = 10000
NTILE = 16
RPT = N // NTILE


def _embed_body(x_ref, w_ref, b_ref, o_ref):
    o_ref[...] = jnp.maximum(
        jnp.dot(x_ref[...], w_ref[...], preferred_element_type=jnp.float32)
        + b_ref[...],
        0.0,
    )


def _embed(x, W, b):
    n, d = x.shape
    h = W.shape[1]
    blk = 2000
    return pl.pallas_call(
        _embed_body,
        grid=(n // blk,),
        in_specs=[
            pl.BlockSpec((blk, d), lambda i: (i, 0)),
            pl.BlockSpec((d, h), lambda i: (0, 0)),
            pl.BlockSpec((1, h), lambda i: (0, 0)),
        ],
        out_specs=pl.BlockSpec((blk, h), lambda i: (i, 0)),
        out_shape=jax.ShapeDtypeStruct((n, h), jnp.float32),
    )(x, W, b.reshape(1, h))


def _spmm_half_kernel(src_ref, r_ref, c_ref, u_ref, acc, gbuf, cbuf, rbuf,
                      is0, is1, is2, is3, gs0, gs1, ss0, ss1):
    c = lax.axis_index("c")
    s = lax.axis_index("s")
    w = c * NTILE + s
    ntot = (r_ref.shape[0] // 128) // 32
    isems = (is0, is1, is2, is3)
    gsems = (gs0, gs1)
    ssems = (ss0, ss1)

    z16 = jnp.zeros((1, 16), jnp.float32)
    zbuf = gbuf.at[0]

    @pl.loop(0, 128)
    def _(i):
        for k8 in range(8):
            zbuf[pl.ds(i, 1), pl.ds(k8 * 16, 16)] = z16

    for j in range(5):
        pltpu.sync_copy(zbuf, acc.at[pl.ds(s * 640 + j * 128, 128)])
    plsc.subcore_barrier()

    def do_idx(k, p):
        off = (w * ntot + k) * 128
        pltpu.sync_copy(c_ref.at[pl.ds(off, 128)], cbuf.at[p])
        pltpu.sync_copy(r_ref.at[pl.ds(off, 128)], rbuf.at[p])

    def do_gather(p):
        pltpu.sync_copy(src_ref.at[cbuf.at[p]], gbuf.at[p])

    def sc_start(p):
        pltpu.async_copy(gbuf.at[p], acc.at[rbuf.at[p]], ssems[p], add=True)

    def sc_wait(p):
        pltpu.make_async_copy(gbuf.at[p], acc.at[rbuf.at[p]],
                              ssems[p]).wait()

    @pl.loop(0, ntot)
    def _(j):
        do_idx(j, 0)
        do_gather(0)
        sc_start(0)
        sc_wait(0)

    plsc.subcore_barrier()
    pltpu.sync_copy(acc.at[pl.ds(s * 640, 640)], u_ref.at[w])


def _spmm(rows, cols, src):
    mesh = plsc.VectorSubcoreMesh(core_axis_name="c", subcore_axis_name="s")
    k = pl.kernel(
        _spmm_half_kernel,
        out_type=jax.ShapeDtypeStruct((2 * NTILE, 640, 128), jnp.float32),
        mesh=mesh,
        scratch_types=[
            pltpu.VMEM_SHARED((NPAD, 128), jnp.float32),
            pltpu.VMEM((2, 128, 128), jnp.float32),
            pltpu.VMEM((4, 128), jnp.int32),
            pltpu.VMEM((4, 128), jnp.int32),
        ] + [pltpu.SemaphoreType.DMA] * 8,
    )
    u = k(src, rows, cols).reshape(2, NPAD, 128)
    return u[0] + u[1]


def _deg_kernel(r1_ref, c1_ref, r2_ref, c2_ref, out_ref,
                a0, a1, a2, a3, ones, ibuf, zbuf):
    c = lax.axis_index("c")
    s = lax.axis_index("s")
    accs = (a0, a1, a2, a3)

    z16 = jnp.zeros((16,), jnp.float32)
    o16 = jnp.full((16,), 1.0, jnp.float32)
    for k8 in range(8):
        ones[pl.ds(k8 * 16, 16)] = o16
    for k8 in range(40):
        zbuf[pl.ds(k8 * 16, 16)] = z16

    lo = s * 640
    for acc in accs:
        pltpu.sync_copy(zbuf, acc.at[pl.ds(lo, 640)])

    plsc.subcore_barrier()

    for m, idx_hbm in enumerate((r1_ref, c1_ref, r2_ref, c2_ref)):
        nch = idx_hbm.shape[0] // 128
        w = c * NTILE + s
        jlo = w * nch // 32
        jhi = (w + 1) * nch // 32

        @pl.loop(jlo, jhi)
        def _(j):
            pltpu.sync_copy(idx_hbm.at[pl.ds(j * 128, 128)], ibuf)
            pltpu.sync_copy(ones, accs[m].at[ibuf], add=True)

    plsc.subcore_barrier()

    for m in range(4):
        pltpu.sync_copy(accs[m].at[pl.ds(lo, 640)],
                        out_ref.at[c * 4 + m, 0, pl.ds(lo, 640)])


NPAD = 10240


def _degrees(r1, c1, r2, c2):
    mesh = plsc.VectorSubcoreMesh(core_axis_name="c", subcore_axis_name="s")
    k = pl.kernel(
        _deg_kernel,
        out_type=jax.ShapeDtypeStruct((8, 1, NPAD), jnp.float32),
        mesh=mesh,
        scratch_types=[
            pltpu.VMEM_SHARED((NPAD,), jnp.float32),
            pltpu.VMEM_SHARED((NPAD,), jnp.float32),
            pltpu.VMEM_SHARED((NPAD,), jnp.float32),
            pltpu.VMEM_SHARED((NPAD,), jnp.float32),
            pltpu.VMEM((128,), jnp.float32),
            pltpu.VMEM((128,), jnp.int32),
            pltpu.VMEM((640,), jnp.float32),
        ],
    )
    d = k(r1, c1, r2, c2).reshape(2, 4, NPAD)[:, :, :N]
    return d[0] + d[1]


def kernel(x, edge_index, edge_index2, W_embed, b_embed, gamma0, beta0,
           W_final, b_final):
    n = x.shape[0]
    r1, c1 = edge_index[0], edge_index[1]
    r2, c2 = edge_index2[0], edge_index2[1]
    deg = _degrees(r1, c1, r2, c2)

    def pad_e(e):
        epad = -e.shape[0] % 16384
        fill = N + (jnp.arange(epad, dtype=jnp.int32) % (NPAD - N))
        return jnp.concatenate([e, fill])

    r1p, c1p, r2p, c2p = pad_e(r1), pad_e(c1), pad_e(r2), pad_e(c2)
    inv = jnp.where(deg > 0, jax.lax.rsqrt(deg), 0.0)
    dr1, dc1, dr2, dc2 = inv[0], inv[1], inv[2], inv[3]
    h = _embed(x, W_embed, b_embed)

    def pass_(y, lo):
        ya = jnp.pad(y[:, lo:lo + 128] * dc1[:, None], ((0, NPAD - n), (0, 0)))
        yb = jnp.pad(y[:, lo:lo + 128] * dc2[:, None], ((0, NPAD - n), (0, 0)))
        u1 = _spmm(r1p, c1p, ya)[:n]
        u2 = _spmm(r2p, c2p, yb)[:n]
        return u1 * dr1[:, None], u2 * dr2[:, None]

    s1h, s2h = pass_(h, 0)
    h1 = jnp.concatenate([s1h, s2h], axis=1)
    m = jnp.mean(h1, axis=0)
    v = jnp.var(h1, axis=0)
    h1 = (h1 - m) / jnp.sqrt(v + 1e-5) * gamma0 + beta0
    v1a, v2a = pass_(h1, 0)
    v1b, v2b = pass_(h1, 128)
    h2 = jnp.concatenate([v1a, v1b, v2a, v2b], axis=1)
    hj = jnp.concatenate([h, h1, h2], axis=1)
    out = hj @ W_final + b_final
    return jax.nn.log_softmax(out, axis=1)


# speedup vs baseline: 2.6475x; 1.2051x over previous
import functools

import jax
import jax.numpy as jnp
from jax import lax
from jax.experimental import pallas as pl
from jax.experimental.pallas import tpu as pltpu
from jax.experimental.pallas import tpu_sc as plsc

N = 10000
NTILE = 16
RPT = N // NTILE


def _embed_body(x_ref, w_ref, b_ref, o_ref):
    o_ref[...] = jnp.maximum(
        jnp.dot(x_ref[...], w_ref[...], preferred_element_type=jnp.float32)
        + b_ref[...],
        0.0,
    )


def _embed(x, W, b):
    n, d = x.shape
    h = W.shape[1]
    blk = 2000
    return pl.pallas_call(
        _embed_body,
        grid=(n // blk,),
        in_specs=[
            pl.BlockSpec((blk, d), lambda i: (i, 0)),
            pl.BlockSpec((d, h), lambda i: (0, 0)),
            pl.BlockSpec((1, h), lambda i: (0, 0)),
        ],
        out_specs=pl.BlockSpec((blk, h), lambda i: (i, 0)),
        out_shape=jax.ShapeDtypeStruct((n, h), jnp.float32),
    )(x, W, b.reshape(1, h))


def _spmm_half_kernel(src_ref, r_ref, c_ref, u_ref, acc, gbuf, cbuf, rbuf,
                      is0, is1, is2, is3, gs0, gs1, ss0, ss1):
    c = lax.axis_index("c")
    s = lax.axis_index("s")
    w = c * NTILE + s
    ntot = (r_ref.shape[0] // 128) // 32
    isems = (is0, is1, is2, is3)
    gsems = (gs0, gs1)
    ssems = (ss0, ss1)

    z16 = jnp.zeros((1, 16), jnp.float32)
    zbuf = gbuf.at[0]

    @pl.loop(0, 128)
    def _(i):
        for k8 in range(8):
            zbuf[pl.ds(i, 1), pl.ds(k8 * 16, 16)] = z16

    for j in range(5):
        pltpu.sync_copy(zbuf, acc.at[pl.ds(s * 640 + j * 128, 128)])
    plsc.subcore_barrier()

    def do_idx(k, p):
        off = (w * ntot + k) * 128
        pltpu.sync_copy(c_ref.at[pl.ds(off, 128)], cbuf.at[p])
        pltpu.sync_copy(r_ref.at[pl.ds(off, 128)], rbuf.at[p])

    def do_gather(p):
        pltpu.sync_copy(src_ref.at[cbuf.at[p]], gbuf.at[p])

    def sc_start(p):
        pltpu.async_copy(gbuf.at[p], acc.at[rbuf.at[p]], ssems[p], add=True)

    def sc_wait(p):
        pltpu.make_async_copy(gbuf.at[p], acc.at[rbuf.at[p]],
                              ssems[p]).wait()

    for p in (0, 1):
        do_idx(p, p)
        do_gather(p)
        sc_start(p)

    @pl.loop(1, ntot // 2)
    def _(i):
        for p in (0, 1):
            sc_wait(p)
            do_idx(2 * i + p, p)
            do_gather(p)
            sc_start(p)

    sc_wait(0)
    sc_wait(1)

    plsc.subcore_barrier()
    pltpu.sync_copy(acc.at[pl.ds(s * 640, 640)], u_ref.at[w])


def _spmm(rows, cols, src):
    mesh = plsc.VectorSubcoreMesh(core_axis_name="c", subcore_axis_name="s")
    k = pl.kernel(
        _spmm_half_kernel,
        out_type=jax.ShapeDtypeStruct((2 * NTILE, 640, 128), jnp.float32),
        mesh=mesh,
        scratch_types=[
            pltpu.VMEM_SHARED((NPAD, 128), jnp.float32),
            pltpu.VMEM((2, 128, 128), jnp.float32),
            pltpu.VMEM((4, 128), jnp.int32),
            pltpu.VMEM((4, 128), jnp.int32),
        ] + [pltpu.SemaphoreType.DMA] * 8,
    )
    u = k(src, rows, cols).reshape(2, NPAD, 128)
    return u[0] + u[1]


def _deg_kernel(r1_ref, c1_ref, r2_ref, c2_ref, out_ref,
                a0, a1, a2, a3, ones, ibuf, zbuf):
    c = lax.axis_index("c")
    s = lax.axis_index("s")
    accs = (a0, a1, a2, a3)

    z16 = jnp.zeros((16,), jnp.float32)
    o16 = jnp.full((16,), 1.0, jnp.float32)
    for k8 in range(8):
        ones[pl.ds(k8 * 16, 16)] = o16
    for k8 in range(40):
        zbuf[pl.ds(k8 * 16, 16)] = z16

    lo = s * 640
    for acc in accs:
        pltpu.sync_copy(zbuf, acc.at[pl.ds(lo, 640)])

    plsc.subcore_barrier()

    for m, idx_hbm in enumerate((r1_ref, c1_ref, r2_ref, c2_ref)):
        nch = idx_hbm.shape[0] // 128
        w = c * NTILE + s
        jlo = w * nch // 32
        jhi = (w + 1) * nch // 32

        @pl.loop(jlo, jhi)
        def _(j):
            pltpu.sync_copy(idx_hbm.at[pl.ds(j * 128, 128)], ibuf)
            pltpu.sync_copy(ones, accs[m].at[ibuf], add=True)

    plsc.subcore_barrier()

    for m in range(4):
        pltpu.sync_copy(accs[m].at[pl.ds(lo, 640)],
                        out_ref.at[c * 4 + m, 0, pl.ds(lo, 640)])


NPAD = 10240


def _degrees(r1, c1, r2, c2):
    mesh = plsc.VectorSubcoreMesh(core_axis_name="c", subcore_axis_name="s")
    k = pl.kernel(
        _deg_kernel,
        out_type=jax.ShapeDtypeStruct((8, 1, NPAD), jnp.float32),
        mesh=mesh,
        scratch_types=[
            pltpu.VMEM_SHARED((NPAD,), jnp.float32),
            pltpu.VMEM_SHARED((NPAD,), jnp.float32),
            pltpu.VMEM_SHARED((NPAD,), jnp.float32),
            pltpu.VMEM_SHARED((NPAD,), jnp.float32),
            pltpu.VMEM((128,), jnp.float32),
            pltpu.VMEM((128,), jnp.int32),
            pltpu.VMEM((640,), jnp.float32),
        ],
    )
    d = k(r1, c1, r2, c2).reshape(2, 4, NPAD)[:, :, :N]
    return d[0] + d[1]


def kernel(x, edge_index, edge_index2, W_embed, b_embed, gamma0, beta0,
           W_final, b_final):
    n = x.shape[0]
    r1, c1 = edge_index[0], edge_index[1]
    r2, c2 = edge_index2[0], edge_index2[1]
    deg = _degrees(r1, c1, r2, c2)

    def pad_e(e):
        epad = -e.shape[0] % 16384
        fill = N + (jnp.arange(epad, dtype=jnp.int32) % (NPAD - N))
        return jnp.concatenate([e, fill])

    r1p, c1p, r2p, c2p = pad_e(r1), pad_e(c1), pad_e(r2), pad_e(c2)
    inv = jnp.where(deg > 0, jax.lax.rsqrt(deg), 0.0)
    dr1, dc1, dr2, dc2 = inv[0], inv[1], inv[2], inv[3]
    h = _embed(x, W_embed, b_embed)

    def pass_(y, lo):
        ya = jnp.pad(y[:, lo:lo + 128] * dc1[:, None], ((0, NPAD - n), (0, 0)))
        yb = jnp.pad(y[:, lo:lo + 128] * dc2[:, None], ((0, NPAD - n), (0, 0)))
        u1 = _spmm(r1p, c1p, ya)[:n]
        u2 = _spmm(r2p, c2p, yb)[:n]
        return u1 * dr1[:, None], u2 * dr2[:, None]

    s1h, s2h = pass_(h, 0)
    h1 = jnp.concatenate([s1h, s2h], axis=1)
    m = jnp.mean(h1, axis=0)
    v = jnp.var(h1, axis=0)
    h1 = (h1 - m) / jnp.sqrt(v + 1e-5) * gamma0 + beta0
    v1a, v2a = pass_(h1, 0)
    v1b, v2b = pass_(h1, 128)
    h2 = jnp.concatenate([v1a, v1b, v2a, v2b], axis=1)
    hj = jnp.concatenate([h, h1, h2], axis=1)
    out = hj @ W_final + b_final
    return jax.nn.log_softmax(out, axis=1)
